# baseline (speedup 1.0000x reference)
import functools
import math
import random

import numpy as np
import jax
import jax.numpy as jnp
from jax import lax
from jax.experimental import pallas as pl
from jax.experimental.pallas import tpu as pltpu
from jax.experimental.pallas import tpu_sc as plsc

B, S, D = 4, 4096, 1024
R = B * S

NC, NS = 2, 16
NW = NC * NS
RPW = R // NW
CH = 32
NCH = RPW // CH


def _static_plan():
    np.random.seed(0)
    random.seed(0)
    alpha = 1.0
    num_seg = 3

    x_len = S
    token_len = math.ceil(x_len / (num_seg - 1))
    sx = int(np.random.randint(int(token_len / 4), int(token_len * 3 / 4)))
    seq_src = []
    for ii in random.sample(range(num_seg), num_seg):
        b1 = int(np.clip(sx + token_len * (ii - 1), 0, x_len))
        b2 = int(np.clip(sx + token_len * ii, 0, x_len))
        seq_src.append(np.arange(b1, b2))
    seq_src = np.concatenate(seq_src)

    lam = float(np.random.beta(alpha, alpha))
    index = np.random.permutation(B)
    cut_len = int(x_len * (1.0 - lam))
    cx = int(np.random.randint(x_len))
    bbx1 = int(np.clip(cx - cut_len // 2, 0, x_len))
    bbx2 = int(np.clip(cx + cut_len // 2, 0, x_len))
    lam_out = 1.0 - (bbx2 - bbx1) / x_len

    src = np.empty((B, S), np.int32)
    for b in range(B):
        src[b, :] = b * S + seq_src
        src[b, bbx1:bbx2] = index[b] * S + seq_src[bbx1:bbx2]
    return src.reshape(-1), np.float32(lam_out), index


_SRC_ROWS, _LAM, _INDEX = _static_plan()
_IDX_NP = np.ascontiguousarray(_SRC_ROWS.reshape(NW, NCH, CH))


def _worker_segments():
    breaks = np.flatnonzero(np.diff(_SRC_ROWS) != 1) + 1
    starts = np.concatenate([[0], breaks])
    ends = np.concatenate([breaks, [R]])
    per_w = [[] for _ in range(NW)]
    for s, e in zip(starts, ends):
        d0, s0, n = int(s), int(_SRC_ROWS[s]), int(e - s)
        while n > 0:
            w = d0 // RPW
            take = min(n, (w + 1) * RPW - d0)
            per_w[w].append((d0, s0, take))
            d0 += take
            s0 += take
            n -= take
    return per_w


_SEGS = _worker_segments()

@functools.lru_cache(maxsize=None)
def _build_gather():
    mesh = plsc.VectorSubcoreMesh(
        core_axis_name="c", subcore_axis_name="s",
        num_cores=NC, num_subcores=NS)

    @functools.partial(
        pl.kernel,
        out_type=jax.ShapeDtypeStruct((R * 8, D // 8), jnp.float32),
        mesh=mesh,
        scratch_types=[
            pltpu.SemaphoreType.DMA,
        ],
    )
    def _copy_rows(x_hbm, out_hbm, sem):
        wid = lax.axis_index("s") * NC + lax.axis_index("c")
        for w in range(NW):
            @pl.when(wid == w)
            def _(w=w):
                handles = [
                    pltpu.async_copy(
                        x_hbm.at[pl.ds(s0 * 8, n * 8)],
                        out_hbm.at[pl.ds(d0 * 8, n * 8)],
                        sem)
                    for d0, s0, n in _SEGS[w]
                ]
                for h in handles:
                    h.wait()

    return _copy_rows


def kernel(x):
    out2d = _build_gather()(x.reshape(R * 8, D // 8))
    mixed_x = out2d.reshape(B, S, D)
    lam = jnp.float32(_LAM)
    index = jnp.asarray(_INDEX, dtype=jnp.int64)
    return (mixed_x, lam, index)


# speedup vs baseline: 31.2603x; 31.2603x over previous
import functools
import math
import random

import numpy as np
import jax
import jax.numpy as jnp
from jax import lax
from jax.experimental import pallas as pl
from jax.experimental.pallas import tpu as pltpu
from jax.experimental.pallas import tpu_sc as plsc

B, S, D = 4, 4096, 1024
R = B * S

NC, NS = 2, 16
NW = NC * NS
RPW = R // NW
CH = 32
NCH = RPW // CH
NBUF = 3


def _static_plan():
    np.random.seed(0)
    random.seed(0)
    alpha = 1.0
    num_seg = 3

    x_len = S
    token_len = math.ceil(x_len / (num_seg - 1))
    sx = int(np.random.randint(int(token_len / 4), int(token_len * 3 / 4)))
    seq_src = []
    for ii in random.sample(range(num_seg), num_seg):
        b1 = int(np.clip(sx + token_len * (ii - 1), 0, x_len))
        b2 = int(np.clip(sx + token_len * ii, 0, x_len))
        seq_src.append(np.arange(b1, b2))
    seq_src = np.concatenate(seq_src)

    lam = float(np.random.beta(alpha, alpha))
    index = np.random.permutation(B)
    cut_len = int(x_len * (1.0 - lam))
    cx = int(np.random.randint(x_len))
    bbx1 = int(np.clip(cx - cut_len // 2, 0, x_len))
    bbx2 = int(np.clip(cx + cut_len // 2, 0, x_len))
    lam_out = 1.0 - (bbx2 - bbx1) / x_len

    src = np.empty((B, S), np.int32)
    for b in range(B):
        src[b, :] = b * S + seq_src
        src[b, bbx1:bbx2] = index[b] * S + seq_src[bbx1:bbx2]
    return src.reshape(-1), np.float32(lam_out), index


_SRC_ROWS, _LAM, _INDEX = _static_plan()
_IDX_NP = np.ascontiguousarray(_SRC_ROWS.reshape(NW, NCH, CH))

@functools.lru_cache(maxsize=None)
def _build_gather():
    mesh = plsc.VectorSubcoreMesh(
        core_axis_name="c", subcore_axis_name="s",
        num_cores=NC, num_subcores=NS)

    @functools.partial(
        pl.kernel,
        out_type=jax.ShapeDtypeStruct((R, D), jnp.float32),
        mesh=mesh,
        scratch_types=(
            [pltpu.VMEM((NCH, CH), jnp.int32)]
            + [pltpu.VMEM((CH, D), jnp.float32) for _ in range(NBUF)]
            + [pltpu.SemaphoreType.DMA for _ in range(2 * NBUF)]
        ),
    )
    def _gather_rows(x_hbm, idx_hbm, out_hbm, idx_v, *scratch):
        bufs = scratch[:NBUF]
        gsems = scratch[NBUF:2 * NBUF]
        ssems = scratch[2 * NBUF:]
        wid = lax.axis_index("s") * NC + lax.axis_index("c")
        base = wid * RPW
        pltpu.sync_copy(idx_hbm.at[wid], idx_v)

        def gather(c):
            k = c % NBUF
            return pltpu.async_copy(x_hbm.at[idx_v.at[c]], bufs[k], gsems[k])

        def scatter(c):
            k = c % NBUF
            return pltpu.async_copy(
                bufs[k], out_hbm.at[pl.ds(base + c * CH, CH)], ssems[k])

        h_g = [None] * NBUF
        h_s = [None] * NBUF
        for c in range(min(NBUF - 1, NCH)):
            h_g[c % NBUF] = gather(c)
        for c in range(NCH):
            g = c + NBUF - 1
            if g < NCH:
                k = g % NBUF
                if h_s[k] is not None:
                    h_s[k].wait()
                h_g[k] = gather(g)
            h_g[c % NBUF].wait()
            h_s[c % NBUF] = scatter(c)
        for k in range(NBUF):
            if h_s[k] is not None:
                h_s[k].wait()

    return _gather_rows


def kernel(x):
    out2d = _build_gather()(x.reshape(R, D), jnp.asarray(_IDX_NP))
    mixed_x = out2d.reshape(B, S, D)
    lam = jnp.float32(_LAM)
    index = jnp.asarray(_INDEX, dtype=jnp.int64)
    return (mixed_x, lam, index)
